# baseline (speedup 1.0000x reference)
import dataclasses
import functools

import jax
import jax.numpy as jnp
from jax import lax
from jax.experimental import pallas as pl
from jax.experimental.pallas import tpu as pltpu
from jax.experimental.pallas import tpu_sc as plsc

N = 50000
D = 128
ED = 16
E = 400000

NB = 16
BS = 3200
NF = NB * BS
E_PAD = 409600
CH = E_PAD // 16
SB = 6400
NSB = CH // SB
G = 64
ROWS_PT = BS // 16
ZROWS = 200
SCALE = 0.08838834764831845

_HIGH = jax.lax.Precision.HIGHEST


def _proj_body(x_ref, wq_ref, bq_ref, wkv_ref, bkv_ref, wsk_ref, bsk_ref,
               q_ref, kv_ref, sk_ref):
    x = x_ref[...]
    q_ref[...] = jnp.dot(x, wq_ref[...], precision=_HIGH,
                         preferred_element_type=jnp.float32) + bq_ref[...]
    kv_ref[...] = jnp.dot(x, wkv_ref[...], precision=_HIGH,
                          preferred_element_type=jnp.float32) + bkv_ref[...]
    sk_ref[...] = jnp.dot(x, wsk_ref[...], precision=_HIGH,
                          preferred_element_type=jnp.float32) + bsk_ref[...]


def _proj(x, wq, bq, wkv, bkv, wsk, bsk):
    BR = 2000
    grid = N // BR
    return pl.pallas_call(
        _proj_body,
        grid=(grid,),
        in_specs=[
            pl.BlockSpec((BR, D), lambda i: (i, 0)),
            pl.BlockSpec((D, D), lambda i: (0, 0)),
            pl.BlockSpec((1, D), lambda i: (0, 0)),
            pl.BlockSpec((D, 256), lambda i: (0, 0)),
            pl.BlockSpec((1, 256), lambda i: (0, 0)),
            pl.BlockSpec((D, D), lambda i: (0, 0)),
            pl.BlockSpec((1, D), lambda i: (0, 0)),
        ],
        out_specs=[
            pl.BlockSpec((BR, D), lambda i: (i, 0)),
            pl.BlockSpec((BR, 256), lambda i: (i, 0)),
            pl.BlockSpec((BR, D), lambda i: (i, 0)),
        ],
        out_shape=[
            jax.ShapeDtypeStruct((N, D), jnp.float32),
            jax.ShapeDtypeStruct((N, 256), jnp.float32),
            jax.ShapeDtypeStruct((N, D), jnp.float32),
        ],
    )(x, wq, bq.reshape(1, -1), wkv, bkv.reshape(1, -1), wsk, bsk.reshape(1, -1))


def _emb_body(ea_ref, we_ref, o_ref):
    o_ref[...] = jnp.dot(ea_ref[...], we_ref[...], precision=_HIGH,
                         preferred_element_type=jnp.float32)


def _emb(ea, we):
    BR = 4000
    grid = E // BR
    return pl.pallas_call(
        _emb_body,
        grid=(grid,),
        in_specs=[
            pl.BlockSpec((BR, ED), lambda i: (i, 0)),
            pl.BlockSpec((ED, D), lambda i: (0, 0)),
        ],
        out_specs=pl.BlockSpec((BR, D), lambda i: (i, 0)),
        out_shape=jax.ShapeDtypeStruct((E, D), jnp.float32),
    )(ea, we)


def _finish_body(numf_ref, denf_ref, sk_ref, o_ref, *, relu):
    den = jnp.sum(denf_ref[...], axis=0)[:, None]
    out = numf_ref[...] / jnp.maximum(den, 1e-16) + sk_ref[...]
    if relu:
        out = jnp.maximum(out, 0.0)
    o_ref[...] = out


def _finish(numf, denf, skip, relu):
    BR = 2048
    grid = pl.cdiv(N, BR)
    return pl.pallas_call(
        functools.partial(_finish_body, relu=relu),
        grid=(grid,),
        in_specs=[
            pl.BlockSpec((BR, D), lambda i: (i, 0)),
            pl.BlockSpec((16, BR), lambda i: (0, i)),
            pl.BlockSpec((BR, D), lambda i: (i, 0)),
        ],
        out_specs=pl.BlockSpec((BR, D), lambda i: (i, 0)),
        out_shape=jax.ShapeDtypeStruct((N, D), jnp.float32),
    )(numf, denf, skip)


_MESH = plsc.VectorSubcoreMesh(core_axis_name="c", subcore_axis_name="s",
                               num_cores=2, num_subcores=16)


def _edge_body(q_hbm, kv_hbm, eae_hbm, src_hbm, dst_hbm, z128_hbm, zf_hbm, zi_hbm,
               numf_hbm, denf_hbm,
               num_acc, den, srcb, dstb, scs, scd, sce,
               qg0, kvg0, eaeg0, qg1, kvg1, eaeg1, dstl,
               sem0, sem1, sem2, sem3, sem4, sem5):
    sc = lax.axis_index("c")
    tid = lax.axis_index("s")
    iota16 = lax.iota(jnp.int32, 16)
    m0 = iota16 == 0

    pltpu.sync_copy(zi_hbm, scs)
    pltpu.sync_copy(zi_hbm, scd)
    pltpu.sync_copy(zi_hbm, sce)

    rowbase = tid * ROWS_PT
    bufs = ((qg0, kvg0, eaeg0, sem0, sem1, sem2),
            (qg1, kvg1, eaeg1, sem3, sem4, sem5))

    def issue(jb, b):
        qg, kvg, eaeg, s0, s1, s2 = bufs[b]
        pltpu.async_copy(kv_hbm.at[scs.at[pl.ds(jb, G)]], kvg, s0)
        pltpu.async_copy(q_hbm.at[scd.at[pl.ds(jb, G)]], qg, s1)
        pltpu.async_copy(eae_hbm.at[sce.at[pl.ds(jb, G)]], eaeg, s2)

    def drain(b):
        qg, kvg, eaeg, s0, s1, s2 = bufs[b]
        pltpu.make_async_copy(kv_hbm.at[scs.at[pl.ds(0, G)]], kvg, s0).wait()
        pltpu.make_async_copy(q_hbm.at[scd.at[pl.ds(0, G)]], qg, s1).wait()
        pltpu.make_async_copy(eae_hbm.at[sce.at[pl.ds(0, G)]], eaeg, s2).wait()

    def compute(jb, b, lo, cnt):
        qg, kvg, eaeg, s0, s1, s2 = bufs[b]
        for s in range(G // 16):
            dv = scd[pl.ds(jb + s * 16, 16)]
            dl = jnp.minimum(jnp.maximum(dv - lo, 0), BS - 1)
            dstl[pl.ds(s * 16, 16)] = dl

        def pe(e, c2):
            acc = qg[e, pl.ds(0, 16)] * (kvg[e, pl.ds(0, 16)] + eaeg[e, pl.ds(0, 16)])
            for h in range(1, 8):
                acc = acc + qg[e, pl.ds(h * 16, 16)] * (
                    kvg[e, pl.ds(h * 16, 16)] + eaeg[e, pl.ds(h * 16, 16)])
            logit = jnp.sum(acc) * SCALE
            valid = (jb + e) < cnt
            ls = jnp.where(valid, logit, jnp.float32(-1e30))
            w = jnp.exp(jnp.full((16,), ls, jnp.float32))
            for h in range(8):
                eaeg[e, pl.ds(h * 16, 16)] = w * (
                    kvg[e, pl.ds(128 + h * 16, 16)] + eaeg[e, pl.ds(h * 16, 16)])
            esplat = jnp.full((16,), e, jnp.int32)
            dls = plsc.load_gather(dstl, [esplat])
            cur = plsc.load_gather(den, [dls])
            plsc.store_scatter(den, [dls], cur + w, mask=m0)
            return c2

        lax.fori_loop(0, G, pe, jnp.int32(0))
        pltpu.sync_copy(eaeg, num_acc.at[dstl], add=True)

    def round_body(r, _ru):
        bucket = sc * (NB // 2) + r
        lo = bucket * BS

        def zero_body(z, _z):
            pltpu.sync_copy(z128_hbm, num_acc.at[pl.ds(rowbase + z * ZROWS, ZROWS)])
            return _z

        lax.fori_loop(0, ROWS_PT // ZROWS, zero_body, jnp.int32(0))
        pltpu.sync_copy(zf_hbm, den)
        plsc.subcore_barrier()

        def sb_body(isb, _sb, lo=lo):
            ebase = tid * CH + isb * SB
            pltpu.sync_copy(src_hbm.at[pl.ds(ebase, SB)], srcb)
            pltpu.sync_copy(dst_hbm.at[pl.ds(ebase, SB)], dstb)

            def scan_body(g, cnt, ebase=ebase, lo=lo):
                off = g * 16
                dv = dstb[pl.ds(off, 16)]
                sv = srcb[pl.ds(off, 16)]
                eid = (ebase + off) + iota16
                m = jnp.logical_and(dv >= lo, dv < lo + BS)
                plsc.store_compressed(scd.at[pl.ds(cnt, 16)], dv, mask=m)
                plsc.store_compressed(scs.at[pl.ds(cnt, 16)], sv, mask=m)
                plsc.store_compressed(sce.at[pl.ds(cnt, 16)], eid, mask=m)
                c = plsc.all_reduce_population_count(m)
                return cnt + jnp.max(c)

            cnt = lax.fori_loop(0, SB // 16, scan_body, jnp.int32(0))
            npairs = (cnt + (2 * G - 1)) >> 7

            @pl.when(npairs > 0)
            def _prime():
                issue(0, 0)
                issue(G, 1)

            def proc(j, carry, lo=lo, cnt=cnt, npairs=npairs):
                jb = 2 * j * G
                more = (j + 1) < npairs
                drain(0)
                compute(jb, 0, lo, cnt)

                @pl.when(more)
                def _i0():
                    issue(jb + 2 * G, 0)

                drain(1)
                compute(jb + G, 1, lo, cnt)

                @pl.when(more)
                def _i1():
                    issue(jb + 3 * G, 1)

                return carry

            lax.fori_loop(0, npairs, proc, jnp.int32(0))
            return _sb

        lax.fori_loop(0, NSB, sb_body, jnp.int32(0))

        plsc.subcore_barrier()

        def flush_body(z, _f, lo=lo):
            pltpu.sync_copy(num_acc.at[pl.ds(rowbase + z * ZROWS, ZROWS)],
                            numf_hbm.at[pl.ds(lo + rowbase + z * ZROWS, ZROWS)])
            return _f

        lax.fori_loop(0, ROWS_PT // ZROWS, flush_body, jnp.int32(0))
        pltpu.sync_copy(den, denf_hbm.at[tid, pl.ds(lo, BS)])
        plsc.subcore_barrier()
        return _ru

    lax.fori_loop(0, NB // 2, round_body, jnp.int32(0))


_SC_PARAMS = pltpu.CompilerParams()
if "needs_layout_passes" in pltpu.CompilerParams.__dataclass_fields__:
    _SC_PARAMS = dataclasses.replace(_SC_PARAMS, needs_layout_passes=False)

_edge_pass = pl.kernel(
    _edge_body,
    compiler_params=_SC_PARAMS,
    out_type=(
        jax.ShapeDtypeStruct((NF, D), jnp.float32),
        jax.ShapeDtypeStruct((16, NF), jnp.float32),
    ),
    mesh=_MESH,
    scratch_types=[
        pltpu.VMEM_SHARED((BS, D), jnp.float32),
        pltpu.VMEM((BS,), jnp.float32),
        pltpu.VMEM((SB,), jnp.int32),
        pltpu.VMEM((SB,), jnp.int32),
        pltpu.VMEM((SB + 16,), jnp.int32),
        pltpu.VMEM((SB + 16,), jnp.int32),
        pltpu.VMEM((SB + 16,), jnp.int32),
        pltpu.VMEM((G, D), jnp.float32),
        pltpu.VMEM((G, 256), jnp.float32),
        pltpu.VMEM((G, D), jnp.float32),
        pltpu.VMEM((G, D), jnp.float32),
        pltpu.VMEM((G, 256), jnp.float32),
        pltpu.VMEM((G, D), jnp.float32),
        pltpu.VMEM((G,), jnp.int32),
        pltpu.SemaphoreType.DMA,
        pltpu.SemaphoreType.DMA,
        pltpu.SemaphoreType.DMA,
        pltpu.SemaphoreType.DMA,
        pltpu.SemaphoreType.DMA,
        pltpu.SemaphoreType.DMA,
    ],
)


def kernel(x_A, x_B, edge_index_A2B, edge_index_B2A, edge_attr_A2B, edge_attr_B2A, params):
    pad_i = jnp.full((E_PAD - E,), 1 << 28, jnp.int32)
    pad_z = jnp.zeros((E_PAD - E,), jnp.int32)
    src_ab = jnp.concatenate([edge_index_A2B[0].astype(jnp.int32), pad_z])
    dst_ab = jnp.concatenate([edge_index_A2B[1].astype(jnp.int32), pad_i])
    src_ba = jnp.concatenate([edge_index_B2A[0].astype(jnp.int32), pad_z])
    dst_ba = jnp.concatenate([edge_index_B2A[1].astype(jnp.int32), pad_i])
    z128 = jnp.zeros((ZROWS, D), jnp.float32)
    zf = jnp.zeros((BS,), jnp.float32)
    zi = jnp.zeros((SB + 16,), jnp.int32)

    for layer in range(2):
        pab = params[layer]["A2B"]
        pba = params[layer]["B2A"]

        q_A, kv_A, skip_A = _proj(
            x_A, pba["Wq"], pba["bq"],
            jnp.concatenate([pab["Wk"], pab["Wv"]], axis=1),
            jnp.concatenate([pab["bk"], pab["bv"]], axis=0),
            pba["Wskip"], pba["bskip"])
        q_B, kv_B, skip_B = _proj(
            x_B, pab["Wq"], pab["bq"],
            jnp.concatenate([pba["Wk"], pba["Wv"]], axis=1),
            jnp.concatenate([pba["bk"], pba["bv"]], axis=0),
            pab["Wskip"], pab["bskip"])

        eae_ab = _emb(edge_attr_A2B, pab["We"])
        eae_ba = _emb(edge_attr_B2A, pba["We"])

        numf_B, denf_B = _edge_pass(q_B, kv_A, eae_ab, src_ab, dst_ab, z128, zf, zi)
        numf_A, denf_A = _edge_pass(q_A, kv_B, eae_ba, src_ba, dst_ba, z128, zf, zi)

        relu = layer == 0
        x_B = _finish(numf_B, denf_B, skip_B, relu)
        x_A = _finish(numf_A, denf_A, skip_A, relu)

    return (x_A, x_B)


# speedup vs baseline: 1.5370x; 1.5370x over previous
import dataclasses
import functools

import jax
import jax.numpy as jnp
from jax import lax
from jax.experimental import pallas as pl
from jax.experimental.pallas import tpu as pltpu
from jax.experimental.pallas import tpu_sc as plsc

N = 50000
D = 128
ED = 16
E = 400000

NB = 10
BS = 5120
NF = NB * BS
E_PAD = 409600
CH = E_PAD // 16
SB = 6400
NSB = CH // SB
G = 64
ROWS_PT = BS // 16
ZROWS = 80
SCALE = 0.08838834764831845

_HIGH = jax.lax.Precision.HIGHEST


def _proj_body(x_ref, wq_ref, bq_ref, wkv_ref, bkv_ref, wsk_ref, bsk_ref,
               q_ref, kv_ref, sk_ref):
    x = x_ref[...]
    q_ref[...] = jnp.dot(x, wq_ref[...], precision=_HIGH,
                         preferred_element_type=jnp.float32) + bq_ref[...]
    kv_ref[...] = jnp.dot(x, wkv_ref[...], precision=_HIGH,
                          preferred_element_type=jnp.float32) + bkv_ref[...]
    sk_ref[...] = jnp.dot(x, wsk_ref[...], precision=_HIGH,
                          preferred_element_type=jnp.float32) + bsk_ref[...]


def _proj(x, wq, bq, wkv, bkv, wsk, bsk):
    BR = 2000
    grid = N // BR
    return pl.pallas_call(
        _proj_body,
        grid=(grid,),
        in_specs=[
            pl.BlockSpec((BR, D), lambda i: (i, 0)),
            pl.BlockSpec((D, D), lambda i: (0, 0)),
            pl.BlockSpec((1, D), lambda i: (0, 0)),
            pl.BlockSpec((D, 256), lambda i: (0, 0)),
            pl.BlockSpec((1, 256), lambda i: (0, 0)),
            pl.BlockSpec((D, D), lambda i: (0, 0)),
            pl.BlockSpec((1, D), lambda i: (0, 0)),
        ],
        out_specs=[
            pl.BlockSpec((BR, D), lambda i: (i, 0)),
            pl.BlockSpec((BR, 256), lambda i: (i, 0)),
            pl.BlockSpec((BR, D), lambda i: (i, 0)),
        ],
        out_shape=[
            jax.ShapeDtypeStruct((N, D), jnp.float32),
            jax.ShapeDtypeStruct((N, 256), jnp.float32),
            jax.ShapeDtypeStruct((N, D), jnp.float32),
        ],
    )(x, wq, bq.reshape(1, -1), wkv, bkv.reshape(1, -1), wsk, bsk.reshape(1, -1))


def _emb_body(ea_ref, we_ref, o_ref):
    o_ref[...] = jnp.dot(ea_ref[...], we_ref[...], precision=_HIGH,
                         preferred_element_type=jnp.float32)


def _emb(ea, we):
    BR = 4000
    grid = E // BR
    return pl.pallas_call(
        _emb_body,
        grid=(grid,),
        in_specs=[
            pl.BlockSpec((BR, ED), lambda i: (i, 0)),
            pl.BlockSpec((ED, D), lambda i: (0, 0)),
        ],
        out_specs=pl.BlockSpec((BR, D), lambda i: (i, 0)),
        out_shape=jax.ShapeDtypeStruct((E, D), jnp.float32),
    )(ea, we)


def _finish_body(numf_ref, denf_ref, sk_ref, o_ref, *, relu):
    den = jnp.sum(denf_ref[...], axis=0)[:, None]
    out = numf_ref[...] / jnp.maximum(den, 1e-16) + sk_ref[...]
    if relu:
        out = jnp.maximum(out, 0.0)
    o_ref[...] = out


def _finish(numf, denf, skip, relu):
    BR = 2048
    grid = pl.cdiv(N, BR)
    return pl.pallas_call(
        functools.partial(_finish_body, relu=relu),
        grid=(grid,),
        in_specs=[
            pl.BlockSpec((BR, D), lambda i: (i, 0)),
            pl.BlockSpec((16, BR), lambda i: (0, i)),
            pl.BlockSpec((BR, D), lambda i: (i, 0)),
        ],
        out_specs=pl.BlockSpec((BR, D), lambda i: (i, 0)),
        out_shape=jax.ShapeDtypeStruct((N, D), jnp.float32),
    )(numf, denf, skip)


_MESH = plsc.VectorSubcoreMesh(core_axis_name="c", subcore_axis_name="s",
                               num_cores=2, num_subcores=16)


def _edge_body(q_hbm, kv_hbm, eae_hbm, src_hbm, dst_hbm, z128_hbm, zf_hbm, zi_hbm,
               numf_hbm, denf_hbm,
               num_acc, den, srcb, dstb, scs, scd, sce,
               qg0, kvg0, eaeg0, dstl,
               sem0, sem1, sem2):
    sc = lax.axis_index("c")
    tid = lax.axis_index("s")
    iota16 = lax.iota(jnp.int32, 16)
    m0 = iota16 == 0

    pltpu.sync_copy(zi_hbm, scs)
    pltpu.sync_copy(zi_hbm, scd)
    pltpu.sync_copy(zi_hbm, sce)

    rowbase = tid * ROWS_PT
    bufs = ((qg0, kvg0, eaeg0, sem0, sem1, sem2),)

    def issue(jb, b):
        qg, kvg, eaeg, s0, s1, s2 = bufs[b]
        pltpu.async_copy(kv_hbm.at[scs.at[pl.ds(jb, G)]], kvg, s0)
        pltpu.async_copy(q_hbm.at[scd.at[pl.ds(jb, G)]], qg, s1)
        pltpu.async_copy(eae_hbm.at[sce.at[pl.ds(jb, G)]], eaeg, s2)

    def drain(b):
        qg, kvg, eaeg, s0, s1, s2 = bufs[b]
        pltpu.make_async_copy(kv_hbm.at[scs.at[pl.ds(0, G)]], kvg, s0).wait()
        pltpu.make_async_copy(q_hbm.at[scd.at[pl.ds(0, G)]], qg, s1).wait()
        pltpu.make_async_copy(eae_hbm.at[sce.at[pl.ds(0, G)]], eaeg, s2).wait()

    def compute(jb, b, lo, cnt):
        qg, kvg, eaeg, s0, s1, s2 = bufs[b]
        for s in range(G // 16):
            dv = scd[pl.ds(jb + s * 16, 16)]
            dl = jnp.minimum(jnp.maximum(dv - lo, 0), BS - 1)
            dstl[pl.ds(s * 16, 16)] = dl

        def pe(e, c2):
            acc = qg[e, pl.ds(0, 16)] * (kvg[e, pl.ds(0, 16)] + eaeg[e, pl.ds(0, 16)])
            for h in range(1, 8):
                acc = acc + qg[e, pl.ds(h * 16, 16)] * (
                    kvg[e, pl.ds(h * 16, 16)] + eaeg[e, pl.ds(h * 16, 16)])
            logit = jnp.sum(acc) * SCALE
            valid = (jb + e) < cnt
            ls = jnp.where(valid, logit, jnp.float32(-1e30))
            w = jnp.exp(jnp.full((16,), ls, jnp.float32))
            for h in range(8):
                eaeg[e, pl.ds(h * 16, 16)] = w * (
                    kvg[e, pl.ds(128 + h * 16, 16)] + eaeg[e, pl.ds(h * 16, 16)])
            esplat = jnp.full((16,), e, jnp.int32)
            dls = plsc.load_gather(dstl, [esplat])
            cur = plsc.load_gather(den, [dls])
            plsc.store_scatter(den, [dls], cur + w, mask=m0)
            return c2

        lax.fori_loop(0, G, pe, jnp.int32(0))
        pltpu.sync_copy(eaeg, num_acc.at[dstl], add=True)

    def round_body(r, _ru):
        bucket = sc * (NB // 2) + r
        lo = bucket * BS

        def zero_body(z, _z):
            pltpu.sync_copy(z128_hbm, num_acc.at[pl.ds(rowbase + z * ZROWS, ZROWS)])
            return _z

        lax.fori_loop(0, ROWS_PT // ZROWS, zero_body, jnp.int32(0))
        pltpu.sync_copy(zf_hbm, den)
        plsc.subcore_barrier()

        def sb_body(isb, _sb, lo=lo):
            ebase = tid * CH + isb * SB
            pltpu.sync_copy(src_hbm.at[pl.ds(ebase, SB)], srcb)
            pltpu.sync_copy(dst_hbm.at[pl.ds(ebase, SB)], dstb)

            def scan_body(g, cnt, ebase=ebase, lo=lo):
                off = g * 16
                dv = dstb[pl.ds(off, 16)]
                sv = srcb[pl.ds(off, 16)]
                eid = (ebase + off) + iota16
                m = jnp.logical_and(dv >= lo, dv < lo + BS)
                plsc.store_compressed(scd.at[pl.ds(cnt, 16)], dv, mask=m)
                plsc.store_compressed(scs.at[pl.ds(cnt, 16)], sv, mask=m)
                plsc.store_compressed(sce.at[pl.ds(cnt, 16)], eid, mask=m)
                c = plsc.all_reduce_population_count(m)
                return cnt + jnp.max(c)

            cnt = lax.fori_loop(0, SB // 16, scan_body, jnp.int32(0))
            ng = (cnt + (G - 1)) >> 6

            def proc(j, carry, lo=lo, cnt=cnt):
                jb = j * G
                issue(jb, 0)
                drain(0)
                compute(jb, 0, lo, cnt)
                return carry

            lax.fori_loop(0, ng, proc, jnp.int32(0))
            return _sb

        lax.fori_loop(0, NSB, sb_body, jnp.int32(0))

        plsc.subcore_barrier()

        def flush_body(z, _f, lo=lo):
            pltpu.sync_copy(num_acc.at[pl.ds(rowbase + z * ZROWS, ZROWS)],
                            numf_hbm.at[pl.ds(lo + rowbase + z * ZROWS, ZROWS)])
            return _f

        lax.fori_loop(0, ROWS_PT // ZROWS, flush_body, jnp.int32(0))
        pltpu.sync_copy(den, denf_hbm.at[tid, pl.ds(lo, BS)])
        plsc.subcore_barrier()
        return _ru

    lax.fori_loop(0, NB // 2, round_body, jnp.int32(0))


_SC_PARAMS = pltpu.CompilerParams()
if "needs_layout_passes" in pltpu.CompilerParams.__dataclass_fields__:
    _SC_PARAMS = dataclasses.replace(_SC_PARAMS, needs_layout_passes=False)

_edge_pass = pl.kernel(
    _edge_body,
    compiler_params=_SC_PARAMS,
    out_type=(
        jax.ShapeDtypeStruct((NF, D), jnp.float32),
        jax.ShapeDtypeStruct((16, NF), jnp.float32),
    ),
    mesh=_MESH,
    scratch_types=[
        pltpu.VMEM_SHARED((BS, D), jnp.float32),
        pltpu.VMEM((BS,), jnp.float32),
        pltpu.VMEM((SB,), jnp.int32),
        pltpu.VMEM((SB,), jnp.int32),
        pltpu.VMEM((SB + 16,), jnp.int32),
        pltpu.VMEM((SB + 16,), jnp.int32),
        pltpu.VMEM((SB + 16,), jnp.int32),
        pltpu.VMEM((G, D), jnp.float32),
        pltpu.VMEM((G, 256), jnp.float32),
        pltpu.VMEM((G, D), jnp.float32),
        pltpu.VMEM((G,), jnp.int32),
        pltpu.SemaphoreType.DMA,
        pltpu.SemaphoreType.DMA,
        pltpu.SemaphoreType.DMA,
    ],
)


def kernel(x_A, x_B, edge_index_A2B, edge_index_B2A, edge_attr_A2B, edge_attr_B2A, params):
    pad_i = jnp.full((E_PAD - E,), 1 << 28, jnp.int32)
    pad_z = jnp.zeros((E_PAD - E,), jnp.int32)
    src_ab = jnp.concatenate([edge_index_A2B[0].astype(jnp.int32), pad_z])
    dst_ab = jnp.concatenate([edge_index_A2B[1].astype(jnp.int32), pad_i])
    src_ba = jnp.concatenate([edge_index_B2A[0].astype(jnp.int32), pad_z])
    dst_ba = jnp.concatenate([edge_index_B2A[1].astype(jnp.int32), pad_i])
    z128 = jnp.zeros((ZROWS, D), jnp.float32)
    zf = jnp.zeros((BS,), jnp.float32)
    zi = jnp.zeros((SB + 16,), jnp.int32)

    for layer in range(2):
        pab = params[layer]["A2B"]
        pba = params[layer]["B2A"]

        q_A, kv_A, skip_A = _proj(
            x_A, pba["Wq"], pba["bq"],
            jnp.concatenate([pab["Wk"], pab["Wv"]], axis=1),
            jnp.concatenate([pab["bk"], pab["bv"]], axis=0),
            pba["Wskip"], pba["bskip"])
        q_B, kv_B, skip_B = _proj(
            x_B, pab["Wq"], pab["bq"],
            jnp.concatenate([pba["Wk"], pba["Wv"]], axis=1),
            jnp.concatenate([pba["bk"], pba["bv"]], axis=0),
            pab["Wskip"], pab["bskip"])

        eae_ab = _emb(edge_attr_A2B, pab["We"])
        eae_ba = _emb(edge_attr_B2A, pba["We"])

        numf_B, denf_B = _edge_pass(q_B, kv_A, eae_ab, src_ab, dst_ab, z128, zf, zi)
        numf_A, denf_A = _edge_pass(q_A, kv_B, eae_ba, src_ba, dst_ba, z128, zf, zi)

        relu = layer == 0
        x_B = _finish(numf_B, denf_B, skip_B, relu)
        x_A = _finish(numf_A, denf_A, skip_A, relu)

    return (x_A, x_B)


# speedup vs baseline: 1.5445x; 1.0049x over previous
import dataclasses
import functools

import jax
import jax.numpy as jnp
from jax import lax
from jax.experimental import pallas as pl
from jax.experimental.pallas import tpu as pltpu
from jax.experimental.pallas import tpu_sc as plsc

N = 50000
D = 128
ED = 16
E = 400000

NB = 10
BS = 5120
NF = NB * BS
E_PAD = 409600
CH = E_PAD // 16
SB = 6400
NSB = CH // SB
G = 64
ROWS_PT = BS // 16
ZROWS = 80
SCALE = 0.08838834764831845

_HIGH = jax.lax.Precision.HIGHEST


def _proj_body(x_ref, wq_ref, bq_ref, wkv_ref, bkv_ref, wsk_ref, bsk_ref,
               q_ref, kv_ref, sk_ref):
    x = x_ref[...]
    q_ref[...] = jnp.dot(x, wq_ref[...], precision=_HIGH,
                         preferred_element_type=jnp.float32) + bq_ref[...]
    kv_ref[...] = jnp.dot(x, wkv_ref[...], precision=_HIGH,
                          preferred_element_type=jnp.float32) + bkv_ref[...]
    sk_ref[...] = jnp.dot(x, wsk_ref[...], precision=_HIGH,
                          preferred_element_type=jnp.float32) + bsk_ref[...]


def _proj(x, wq, bq, wkv, bkv, wsk, bsk):
    BR = 2000
    grid = N // BR
    return pl.pallas_call(
        _proj_body,
        grid=(grid,),
        in_specs=[
            pl.BlockSpec((BR, D), lambda i: (i, 0)),
            pl.BlockSpec((D, D), lambda i: (0, 0)),
            pl.BlockSpec((1, D), lambda i: (0, 0)),
            pl.BlockSpec((D, 256), lambda i: (0, 0)),
            pl.BlockSpec((1, 256), lambda i: (0, 0)),
            pl.BlockSpec((D, D), lambda i: (0, 0)),
            pl.BlockSpec((1, D), lambda i: (0, 0)),
        ],
        out_specs=[
            pl.BlockSpec((BR, D), lambda i: (i, 0)),
            pl.BlockSpec((BR, 256), lambda i: (i, 0)),
            pl.BlockSpec((BR, D), lambda i: (i, 0)),
        ],
        out_shape=[
            jax.ShapeDtypeStruct((N, D), jnp.float32),
            jax.ShapeDtypeStruct((N, 256), jnp.float32),
            jax.ShapeDtypeStruct((N, D), jnp.float32),
        ],
    )(x, wq, bq.reshape(1, -1), wkv, bkv.reshape(1, -1), wsk, bsk.reshape(1, -1))


def _emb_body(ea_ref, we_ref, o_ref):
    o_ref[...] = jnp.dot(ea_ref[...], we_ref[...], precision=_HIGH,
                         preferred_element_type=jnp.float32)


def _emb(ea, we):
    BR = 4000
    grid = E // BR
    return pl.pallas_call(
        _emb_body,
        grid=(grid,),
        in_specs=[
            pl.BlockSpec((BR, ED), lambda i: (i, 0)),
            pl.BlockSpec((ED, D), lambda i: (0, 0)),
        ],
        out_specs=pl.BlockSpec((BR, D), lambda i: (i, 0)),
        out_shape=jax.ShapeDtypeStruct((E, D), jnp.float32),
    )(ea, we)


def _finish_body(numf_ref, denf_ref, sk_ref, o_ref, *, relu):
    den = jnp.sum(denf_ref[...], axis=0)[:, None]
    out = numf_ref[...] / jnp.maximum(den, 1e-16) + sk_ref[...]
    if relu:
        out = jnp.maximum(out, 0.0)
    o_ref[...] = out


def _finish(numf, denf, skip, relu):
    BR = 2048
    grid = pl.cdiv(N, BR)
    return pl.pallas_call(
        functools.partial(_finish_body, relu=relu),
        grid=(grid,),
        in_specs=[
            pl.BlockSpec((BR, D), lambda i: (i, 0)),
            pl.BlockSpec((16, BR), lambda i: (0, i)),
            pl.BlockSpec((BR, D), lambda i: (i, 0)),
        ],
        out_specs=pl.BlockSpec((BR, D), lambda i: (i, 0)),
        out_shape=jax.ShapeDtypeStruct((N, D), jnp.float32),
    )(numf, denf, skip)


_MESH = plsc.VectorSubcoreMesh(core_axis_name="c", subcore_axis_name="s",
                               num_cores=2, num_subcores=16)


def _edge_body(q_hbm, kv_hbm, eae_hbm, src_hbm, dst_hbm, z128_hbm, zf_hbm, zi_hbm,
               numf_hbm, denf_hbm,
               num_acc, den, srcb, dstb, scs, scd, sce,
               qg0, kvg0, eaeg0, dstl,
               sem0, sem1, sem2):
    sc = lax.axis_index("c")
    tid = lax.axis_index("s")
    iota16 = lax.iota(jnp.int32, 16)
    m0 = iota16 == 0

    pltpu.sync_copy(zi_hbm, scs)
    pltpu.sync_copy(zi_hbm, scd)
    pltpu.sync_copy(zi_hbm, sce)

    rowbase = tid * ROWS_PT
    bufs = ((qg0, kvg0, eaeg0, sem0, sem1, sem2),)

    def issue(jb, b):
        qg, kvg, eaeg, s0, s1, s2 = bufs[b]
        pltpu.async_copy(kv_hbm.at[scs.at[pl.ds(jb, G)]], kvg, s0)
        pltpu.async_copy(q_hbm.at[scd.at[pl.ds(jb, G)]], qg, s1)
        pltpu.async_copy(eae_hbm.at[sce.at[pl.ds(jb, G)]], eaeg, s2)

    def drain(b):
        qg, kvg, eaeg, s0, s1, s2 = bufs[b]
        pltpu.make_async_copy(kv_hbm.at[scs.at[pl.ds(0, G)]], kvg, s0).wait()
        pltpu.make_async_copy(q_hbm.at[scd.at[pl.ds(0, G)]], qg, s1).wait()
        pltpu.make_async_copy(eae_hbm.at[sce.at[pl.ds(0, G)]], eaeg, s2).wait()

    def compute(jb, b, lo, cnt):
        qg, kvg, eaeg, s0, s1, s2 = bufs[b]
        for s in range(G // 16):
            dv = scd[pl.ds(jb + s * 16, 16)]
            dl = jnp.minimum(jnp.maximum(dv - lo, 0), BS - 1)
            dstl[pl.ds(s * 16, 16)] = dl

        def pe(e, c2):
            acc = qg[e, pl.ds(0, 16)] * (kvg[e, pl.ds(0, 16)] + eaeg[e, pl.ds(0, 16)])
            for h in range(1, 8):
                acc = acc + qg[e, pl.ds(h * 16, 16)] * (
                    kvg[e, pl.ds(h * 16, 16)] + eaeg[e, pl.ds(h * 16, 16)])
            logit = jnp.sum(acc) * SCALE
            valid = (jb + e) < cnt
            ls = jnp.where(valid, logit, jnp.float32(-1e30))
            w = jnp.exp(jnp.full((16,), ls, jnp.float32))
            for h in range(8):
                eaeg[e, pl.ds(h * 16, 16)] = w * (
                    kvg[e, pl.ds(128 + h * 16, 16)] + eaeg[e, pl.ds(h * 16, 16)])
            esplat = jnp.full((16,), e, jnp.int32)
            dls = plsc.load_gather(dstl, [esplat])
            cur = plsc.load_gather(den, [dls])
            plsc.store_scatter(den, [dls], cur + w, mask=m0)
            return c2

        lax.fori_loop(0, G, pe, jnp.int32(0), unroll=2)
        pltpu.sync_copy(eaeg, num_acc.at[dstl], add=True)

    def round_body(r, _ru):
        bucket = sc * (NB // 2) + r
        lo = bucket * BS

        def zero_body(z, _z):
            pltpu.sync_copy(z128_hbm, num_acc.at[pl.ds(rowbase + z * ZROWS, ZROWS)])
            return _z

        lax.fori_loop(0, ROWS_PT // ZROWS, zero_body, jnp.int32(0))
        pltpu.sync_copy(zf_hbm, den)
        plsc.subcore_barrier()

        def sb_body(isb, _sb, lo=lo):
            ebase = tid * CH + isb * SB
            pltpu.sync_copy(src_hbm.at[pl.ds(ebase, SB)], srcb)
            pltpu.sync_copy(dst_hbm.at[pl.ds(ebase, SB)], dstb)

            def scan_body(g, cnt, ebase=ebase, lo=lo):
                off = g * 16
                dv = dstb[pl.ds(off, 16)]
                sv = srcb[pl.ds(off, 16)]
                eid = (ebase + off) + iota16
                m = jnp.logical_and(dv >= lo, dv < lo + BS)
                plsc.store_compressed(scd.at[pl.ds(cnt, 16)], dv, mask=m)
                plsc.store_compressed(scs.at[pl.ds(cnt, 16)], sv, mask=m)
                plsc.store_compressed(sce.at[pl.ds(cnt, 16)], eid, mask=m)
                c = plsc.all_reduce_population_count(m)
                return cnt + jnp.max(c)

            cnt = lax.fori_loop(0, SB // 16, scan_body, jnp.int32(0))
            ng = (cnt + (G - 1)) >> 6

            def proc(j, carry, lo=lo, cnt=cnt):
                jb = j * G
                issue(jb, 0)
                drain(0)
                compute(jb, 0, lo, cnt)
                return carry

            lax.fori_loop(0, ng, proc, jnp.int32(0))
            return _sb

        lax.fori_loop(0, NSB, sb_body, jnp.int32(0))

        plsc.subcore_barrier()

        def flush_body(z, _f, lo=lo):
            pltpu.sync_copy(num_acc.at[pl.ds(rowbase + z * ZROWS, ZROWS)],
                            numf_hbm.at[pl.ds(lo + rowbase + z * ZROWS, ZROWS)])
            return _f

        lax.fori_loop(0, ROWS_PT // ZROWS, flush_body, jnp.int32(0))
        pltpu.sync_copy(den, denf_hbm.at[tid, pl.ds(lo, BS)])
        plsc.subcore_barrier()
        return _ru

    lax.fori_loop(0, NB // 2, round_body, jnp.int32(0))


_SC_PARAMS = pltpu.CompilerParams()
if "needs_layout_passes" in pltpu.CompilerParams.__dataclass_fields__:
    _SC_PARAMS = dataclasses.replace(_SC_PARAMS, needs_layout_passes=False)

_edge_pass = pl.kernel(
    _edge_body,
    compiler_params=_SC_PARAMS,
    out_type=(
        jax.ShapeDtypeStruct((NF, D), jnp.float32),
        jax.ShapeDtypeStruct((16, NF), jnp.float32),
    ),
    mesh=_MESH,
    scratch_types=[
        pltpu.VMEM_SHARED((BS, D), jnp.float32),
        pltpu.VMEM((BS,), jnp.float32),
        pltpu.VMEM((SB,), jnp.int32),
        pltpu.VMEM((SB,), jnp.int32),
        pltpu.VMEM((SB + 16,), jnp.int32),
        pltpu.VMEM((SB + 16,), jnp.int32),
        pltpu.VMEM((SB + 16,), jnp.int32),
        pltpu.VMEM((G, D), jnp.float32),
        pltpu.VMEM((G, 256), jnp.float32),
        pltpu.VMEM((G, D), jnp.float32),
        pltpu.VMEM((G,), jnp.int32),
        pltpu.SemaphoreType.DMA,
        pltpu.SemaphoreType.DMA,
        pltpu.SemaphoreType.DMA,
    ],
)


def kernel(x_A, x_B, edge_index_A2B, edge_index_B2A, edge_attr_A2B, edge_attr_B2A, params):
    pad_i = jnp.full((E_PAD - E,), 1 << 28, jnp.int32)
    pad_z = jnp.zeros((E_PAD - E,), jnp.int32)
    src_ab = jnp.concatenate([edge_index_A2B[0].astype(jnp.int32), pad_z])
    dst_ab = jnp.concatenate([edge_index_A2B[1].astype(jnp.int32), pad_i])
    src_ba = jnp.concatenate([edge_index_B2A[0].astype(jnp.int32), pad_z])
    dst_ba = jnp.concatenate([edge_index_B2A[1].astype(jnp.int32), pad_i])
    z128 = jnp.zeros((ZROWS, D), jnp.float32)
    zf = jnp.zeros((BS,), jnp.float32)
    zi = jnp.zeros((SB + 16,), jnp.int32)

    for layer in range(2):
        pab = params[layer]["A2B"]
        pba = params[layer]["B2A"]

        q_A, kv_A, skip_A = _proj(
            x_A, pba["Wq"], pba["bq"],
            jnp.concatenate([pab["Wk"], pab["Wv"]], axis=1),
            jnp.concatenate([pab["bk"], pab["bv"]], axis=0),
            pba["Wskip"], pba["bskip"])
        q_B, kv_B, skip_B = _proj(
            x_B, pab["Wq"], pab["bq"],
            jnp.concatenate([pba["Wk"], pba["Wv"]], axis=1),
            jnp.concatenate([pba["bk"], pba["bv"]], axis=0),
            pab["Wskip"], pab["bskip"])

        eae_ab = _emb(edge_attr_A2B, pab["We"])
        eae_ba = _emb(edge_attr_B2A, pba["We"])

        numf_B, denf_B = _edge_pass(q_B, kv_A, eae_ab, src_ab, dst_ab, z128, zf, zi)
        numf_A, denf_A = _edge_pass(q_A, kv_B, eae_ba, src_ba, dst_ba, z128, zf, zi)

        relu = layer == 0
        x_B = _finish(numf_B, denf_B, skip_B, relu)
        x_A = _finish(numf_A, denf_A, skip_A, relu)

    return (x_A, x_B)
